# Initial kernel scaffold; baseline (speedup 1.0000x reference)
import functools

import jax
import jax.numpy as jnp
from jax import lax
from jax.experimental import pallas as pl
from jax.experimental.pallas import tpu as pltpu
from jax.experimental.pallas import tpu_sc as plsc

N = 10000
E = 320000
D = 128

NPAD = 10240
SLICE = NPAD // 16
CHUNK = 128
NUM_CHUNKS = E // CHUNK
NW = 32
ITERS = -(-NUM_CHUNKS // NW)

_mesh = plsc.VectorSubcoreMesh(core_axis_name="c", subcore_axis_name="s")


@functools.partial(
    pl.kernel,
    mesh=_mesh,
    out_type=jax.ShapeDtypeStruct((2, NPAD), jnp.float32),
    scratch_types=[
        pltpu.VMEM((CHUNK,), jnp.int32),
        pltpu.VMEM((CHUNK,), jnp.float32),
        pltpu.VMEM_SHARED((NPAD,), jnp.float32),
    ],
)
def _sc_degree(rows_hbm, zeros1_hbm, deg_out, idx_v, ones_v, deg_sh):
    c = lax.axis_index("c")
    s = lax.axis_index("s")
    wid = s * 2 + c
    for i in range(CHUNK // 16):
        ones_v[pl.ds(i * 16, 16)] = jnp.ones((16,), jnp.float32)
    pltpu.sync_copy(zeros1_hbm.at[pl.ds(s * SLICE, SLICE)],
                    deg_sh.at[pl.ds(s * SLICE, SLICE)])
    plsc.subcore_barrier()

    def body(i, carry):
        chunk = wid + NW * i

        @pl.when(chunk < NUM_CHUNKS)
        def _():
            pltpu.sync_copy(rows_hbm.at[pl.ds(chunk * CHUNK, CHUNK)], idx_v)
            pltpu.sync_copy(ones_v, deg_sh.at[idx_v], add=True)

        return carry

    lax.fori_loop(0, ITERS, body, 0)
    plsc.subcore_barrier()
    pltpu.sync_copy(deg_sh.at[pl.ds(s * SLICE, SLICE)],
                    deg_out.at[c, pl.ds(s * SLICE, SLICE)])


@functools.partial(
    pl.kernel,
    mesh=_mesh,
    out_type=jax.ShapeDtypeStruct((2, NPAD, D), jnp.float32),
    scratch_types=[
        pltpu.VMEM((CHUNK,), jnp.int32),
        pltpu.VMEM((CHUNK,), jnp.int32),
        pltpu.VMEM((CHUNK, D), jnp.float32),
        pltpu.VMEM_SHARED((NPAD, D), jnp.float32),
        pltpu.SemaphoreType.DMA,
    ],
)
def _sc_aggregate(u_hbm, cols_hbm, rows_hbm, zeros2_hbm, s_out,
                  cid_v, rid_v, rows_v, s_sh, sem):
    c = lax.axis_index("c")
    s = lax.axis_index("s")
    wid = s * 2 + c
    pltpu.sync_copy(zeros2_hbm.at[pl.ds(s * SLICE, SLICE)],
                    s_sh.at[pl.ds(s * SLICE, SLICE)])
    plsc.subcore_barrier()

    def body(i, carry):
        chunk = wid + NW * i

        @pl.when(chunk < NUM_CHUNKS)
        def _():
            pltpu.sync_copy(cols_hbm.at[pl.ds(chunk * CHUNK, CHUNK)], cid_v)
            pltpu.sync_copy(rows_hbm.at[pl.ds(chunk * CHUNK, CHUNK)], rid_v)
            pltpu.async_copy(u_hbm.at[cid_v], rows_v, sem).wait()
            pltpu.sync_copy(rows_v, s_sh.at[rid_v], add=True)

        return carry

    lax.fori_loop(0, ITERS, body, 0)
    plsc.subcore_barrier()
    pltpu.sync_copy(s_sh.at[pl.ds(s * SLICE, SLICE)],
                    s_out.at[c, pl.ds(s * SLICE, SLICE)])


def _tc_scale_body(deg_ref, x_ref, u_ref, dinv_ref):
    deg = deg_ref[0, :] + deg_ref[1, :]
    dinv = lax.rsqrt(deg)
    dinv_ref[:, 0] = dinv
    u_ref[...] = dinv[:, None] * x_ref[...]


def _tc_final_body(s_ref, u_ref, dinv_ref, w_ref, b_ref, out_ref):
    agg = s_ref[0] + s_ref[1] + u_ref[...]
    h = dinv_ref[:, 0][:, None] * agg
    hw = lax.dot_general(h, w_ref[...], (((1,), (1,)), ((), ())),
                         preferred_element_type=jnp.float32)
    out_ref[...] = jnp.maximum(hw + b_ref[...], 0.0)


BLK = 1250
GRID = N // BLK


def kernel(x, edge_index, W, b):
    rows = edge_index[0]
    cols = edge_index[1]
    zeros1 = jnp.zeros((NPAD,), jnp.float32)
    zeros2 = jnp.zeros((NPAD, D), jnp.float32)

    deg_parts = _sc_degree(rows, zeros1)

    u, dinv = pl.pallas_call(
        _tc_scale_body,
        grid=(GRID,),
        in_specs=[
            pl.BlockSpec((2, BLK), lambda i: (0, i)),
            pl.BlockSpec((BLK, D), lambda i: (i, 0)),
        ],
        out_specs=[
            pl.BlockSpec((BLK, D), lambda i: (i, 0)),
            pl.BlockSpec((BLK, 1), lambda i: (i, 0)),
        ],
        out_shape=[
            jax.ShapeDtypeStruct((N, D), jnp.float32),
            jax.ShapeDtypeStruct((N, 1), jnp.float32),
        ],
    )(deg_parts, x)

    s_parts = _sc_aggregate(u, cols, rows, zeros2)

    out = pl.pallas_call(
        _tc_final_body,
        grid=(GRID,),
        in_specs=[
            pl.BlockSpec((2, BLK, D), lambda i: (0, i, 0)),
            pl.BlockSpec((BLK, D), lambda i: (i, 0)),
            pl.BlockSpec((BLK, 1), lambda i: (i, 0)),
            pl.BlockSpec((D, D), lambda i: (0, 0)),
            pl.BlockSpec((1, D), lambda i: (0, 0)),
        ],
        out_specs=pl.BlockSpec((BLK, D), lambda i: (i, 0)),
        out_shape=jax.ShapeDtypeStruct((N, D), jnp.float32),
    )(s_parts, u, dinv, W, b.reshape(1, D))

    return out


# speedup vs baseline: 14.7002x; 14.7002x over previous
import functools

import jax
import jax.numpy as jnp
from jax import lax
from jax.experimental import pallas as pl
from jax.experimental.pallas import tpu as pltpu
from jax.experimental.pallas import tpu_sc as plsc

N = 10000
E = 320000
D = 128

NPAD = 10240
SLICE = NPAD // 16
CHUNK = 128
NUM_CHUNKS = E // CHUNK
NW = 32
ITERS = -(-NUM_CHUNKS // NW)

_mesh = plsc.VectorSubcoreMesh(core_axis_name="c", subcore_axis_name="s")


@functools.partial(
    pl.kernel,
    mesh=_mesh,
    out_type=jax.ShapeDtypeStruct((2, NPAD), jnp.float32),
    scratch_types=[
        pltpu.VMEM((CHUNK,), jnp.int32),
        pltpu.VMEM((CHUNK,), jnp.float32),
        pltpu.VMEM_SHARED((NPAD,), jnp.float32),
    ],
)
def _sc_degree(rows_hbm, zeros1_hbm, deg_out, idx_v, ones_v, deg_sh):
    c = lax.axis_index("c")
    s = lax.axis_index("s")
    wid = s * 2 + c
    for i in range(CHUNK // 16):
        ones_v[pl.ds(i * 16, 16)] = jnp.ones((16,), jnp.float32)
    pltpu.sync_copy(zeros1_hbm.at[pl.ds(s * SLICE, SLICE)],
                    deg_sh.at[pl.ds(s * SLICE, SLICE)])
    plsc.subcore_barrier()

    def body(i, carry):
        chunk = wid + NW * i

        @pl.when(chunk < NUM_CHUNKS)
        def _():
            pltpu.sync_copy(rows_hbm.at[pl.ds(chunk * CHUNK, CHUNK)], idx_v)
            pltpu.sync_copy(ones_v, deg_sh.at[idx_v], add=True)

        return carry

    lax.fori_loop(0, ITERS, body, 0)
    plsc.subcore_barrier()
    pltpu.sync_copy(deg_sh.at[pl.ds(s * SLICE, SLICE)],
                    deg_out.at[c, pl.ds(s * SLICE, SLICE)])


@functools.partial(
    pl.kernel,
    mesh=_mesh,
    out_type=jax.ShapeDtypeStruct((2, NPAD, D), jnp.float32),
    scratch_types=[
        pltpu.VMEM((CHUNK,), jnp.int32),
        pltpu.VMEM((CHUNK,), jnp.int32),
        pltpu.VMEM((CHUNK, D), jnp.float32),
        pltpu.VMEM_SHARED((NPAD, D), jnp.float32),
        pltpu.SemaphoreType.DMA,
    ],
)
def _sc_aggregate(u_hbm, cols_hbm, rows_hbm, zeros2_hbm, s_out,
                  cid_v, rid_v, rows_v, s_sh, sem):
    c = lax.axis_index("c")
    s = lax.axis_index("s")
    wid = s * 2 + c
    pltpu.sync_copy(zeros2_hbm.at[pl.ds(s * SLICE, SLICE)],
                    s_sh.at[pl.ds(s * SLICE, SLICE)])
    plsc.subcore_barrier()

    def body(i, carry):
        chunk = wid + NW * i

        @pl.when(chunk < NUM_CHUNKS)
        def _():
            pltpu.sync_copy(cols_hbm.at[pl.ds(chunk * CHUNK, CHUNK)], cid_v)
            pltpu.sync_copy(rows_hbm.at[pl.ds(chunk * CHUNK, CHUNK)], rid_v)
            pltpu.async_copy(u_hbm.at[cid_v], rows_v, sem).wait()
            pltpu.sync_copy(rows_v, s_sh.at[rid_v], add=True)

        return carry

    lax.fori_loop(0, ITERS, body, 0)
    plsc.subcore_barrier()
    pltpu.sync_copy(s_sh.at[pl.ds(s * SLICE, SLICE)],
                    s_out.at[c, pl.ds(s * SLICE, SLICE)])


def _tc_scale_body(deg_ref, x_ref, u_ref, dinv_ref):
    deg = deg_ref[0] + deg_ref[1]
    dinv = lax.rsqrt(deg)
    dinv_ref[...] = dinv
    u_ref[...] = dinv * x_ref[...]


def _tc_final_body(s_ref, u_ref, dinv_ref, w_ref, b_ref, out_ref):
    agg = s_ref[0] + s_ref[1] + u_ref[...]
    h = dinv_ref[...] * agg
    hw = lax.dot_general(h, w_ref[...], (((1,), (1,)), ((), ())),
                         preferred_element_type=jnp.float32)
    out_ref[...] = jnp.maximum(hw + b_ref[...], 0.0)


BLK = 2000
GRID = N // BLK


def kernel(x, edge_index, W, b):
    rows = edge_index[0]
    cols = edge_index[1]
    zeros1 = jnp.zeros((NPAD,), jnp.float32)
    zeros2 = jnp.zeros((NPAD, D), jnp.float32)

    deg_parts = _sc_degree(rows, zeros1).reshape(2, NPAD, 1)

    u, dinv = pl.pallas_call(
        _tc_scale_body,
        grid=(GRID,),
        in_specs=[
            pl.BlockSpec((2, BLK, 1), lambda i: (0, i, 0)),
            pl.BlockSpec((BLK, D), lambda i: (i, 0)),
        ],
        out_specs=[
            pl.BlockSpec((BLK, D), lambda i: (i, 0)),
            pl.BlockSpec((BLK, 1), lambda i: (i, 0)),
        ],
        out_shape=[
            jax.ShapeDtypeStruct((N, D), jnp.float32),
            jax.ShapeDtypeStruct((N, 1), jnp.float32),
        ],
    )(deg_parts, x)

    s_parts = _sc_aggregate(u, cols, rows, zeros2)

    out = pl.pallas_call(
        _tc_final_body,
        grid=(GRID,),
        in_specs=[
            pl.BlockSpec((2, BLK, D), lambda i: (0, i, 0)),
            pl.BlockSpec((BLK, D), lambda i: (i, 0)),
            pl.BlockSpec((BLK, 1), lambda i: (i, 0)),
            pl.BlockSpec((D, D), lambda i: (0, 0)),
            pl.BlockSpec((1, D), lambda i: (0, 0)),
        ],
        out_specs=pl.BlockSpec((BLK, D), lambda i: (i, 0)),
        out_shape=jax.ShapeDtypeStruct((N, D), jnp.float32),
    )(s_parts, u, dinv, W, b.reshape(1, D))

    return out
